# baseline (speedup 1.0000x reference)
import functools

import jax
import jax.numpy as jnp
from jax import lax
from jax.experimental import pallas as pl
from jax.experimental.pallas import tpu as pltpu
from jax.experimental.pallas import tpu_sc as plsc

N = 10000
E = 320000
D = 128
R = 32
NB = 8

NC = 2
NS = 16
NW = NC * NS
EW = E // NW
CH = 128
NCH = 79
EWP = NCH * CH
LANES = 16
NP = 10240
SR = NP // NS
DC = 40


def _wr(comp_smem, basis_ref, r):
    acc = comp_smem[r, 0] * basis_ref[0]
    for b in range(1, NB):
        acc = acc + comp_smem[r, b] * basis_ref[b]
    return acc


def _xall1_body(comp_smem, x_ref, basis_ref, t_ref, s_ref, out_ref, g_ref):
    r = pl.program_id(0)

    @pl.when(r == 0)
    def _():
        g_ref[...] = t_ref[...] * N + s_ref[...]

    out_ref[0] = jnp.dot(x_ref[...], _wr(comp_smem, basis_ref, r),
                         preferred_element_type=jnp.float32)


def _xall1(x, basis, comp, t2, s2):
    return pl.pallas_call(
        _xall1_body,
        grid=(R,),
        in_specs=[
            pl.BlockSpec(memory_space=pltpu.SMEM),
            pl.BlockSpec((N, D), lambda r: (0, 0)),
            pl.BlockSpec((NB, D, D), lambda r: (0, 0, 0)),
            pl.BlockSpec((NW * NCH, CH), lambda r: (0, 0)),
            pl.BlockSpec((NW * NCH, CH), lambda r: (0, 0)),
        ],
        out_specs=[
            pl.BlockSpec((1, N, D), lambda r: (r, 0, 0)),
            pl.BlockSpec((NW * NCH, CH), lambda r: (0, 0)),
        ],
        out_shape=[
            jax.ShapeDtypeStruct((R, N, D), jnp.float32),
            jax.ShapeDtypeStruct((NW * NCH, CH), jnp.int32),
        ],
    )(comp, x, basis, t2, s2)


def _xall2_body(comp_smem, agga_ref, aggb_ref, x0_ref, root_ref, bias_ref,
                basis_ref, out_ref, x1_ref, x1s_ref):
    r = pl.program_id(0)

    @pl.when(r == 0)
    def _():
        x1 = jnp.tanh(agga_ref[0:N] + aggb_ref[0:N] + bias_ref[...]
                      + jnp.dot(x0_ref[...], root_ref[...],
                                preferred_element_type=jnp.float32))
        x1s_ref[...] = x1
        x1_ref[...] = x1

    out_ref[0] = jnp.dot(x1s_ref[...], _wr(comp_smem, basis_ref, r),
                         preferred_element_type=jnp.float32)


def _xall2(agg1, x0, root1, bias1, basis, comp):
    return pl.pallas_call(
        _xall2_body,
        grid=(R,),
        in_specs=[
            pl.BlockSpec(memory_space=pltpu.SMEM),
            pl.BlockSpec((NP, D), lambda r: (0, 0)),
            pl.BlockSpec((NP, D), lambda r: (1, 0)),
            pl.BlockSpec((N, D), lambda r: (0, 0)),
            pl.BlockSpec((D, D), lambda r: (0, 0)),
            pl.BlockSpec((1, D), lambda r: (0, 0)),
            pl.BlockSpec((NB, D, D), lambda r: (0, 0, 0)),
        ],
        out_specs=[
            pl.BlockSpec((1, N, D), lambda r: (r, 0, 0)),
            pl.BlockSpec((N, D), lambda r: (0, 0)),
        ],
        out_shape=[
            jax.ShapeDtypeStruct((R, N, D), jnp.float32),
            jax.ShapeDtypeStruct((N, D), jnp.float32),
        ],
        scratch_shapes=[pltpu.VMEM((N, D), jnp.float32)],
    )(comp, agg1, agg1, x0, root1, bias1.reshape(1, D), basis)


PASS_SPLITS = ((0, 24), (24, 24), (48, 24), (72, 7))
PB = 24


def _sc_body(g_hbm, d_hbm, xall_hbm, out_hbm,
             gbuf, dbuf, gcur0, gcur1, dcur0, dcur1, rows0, rows1,
             zbuf, agg, sem0, sem1):
    cid = lax.axis_index("c")
    sid = lax.axis_index("s")
    wid = cid * NS + sid

    def _zr(i, carry):
        for j in range(D // LANES):
            zbuf[i, pl.ds(j * LANES, LANES)] = jnp.zeros((LANES,), jnp.float32)
        return carry

    lax.fori_loop(0, DC, _zr, 0)
    for k in range(SR // DC):
        pltpu.sync_copy(zbuf, agg.at[pl.ds(sid * SR + k * DC, DC)])
    plsc.subcore_barrier()

    def _stage_idx(c, gcur, dcur):
        for j in range(CH // LANES):
            sl = pl.ds(j * LANES, LANES)
            gcur[sl] = gbuf[c, sl]
            dcur[sl] = dbuf[c, sl]

    def _gather(gcur, buf, sem):
        pltpu.make_async_copy(xall_hbm.at[gcur], buf, sem).start()

    def _drain(gcur, buf, sem):
        pltpu.make_async_copy(xall_hbm.at[gcur], buf, sem).wait()

    def _scatter(dcur, buf):
        pltpu.sync_copy(buf, agg.at[dcur], add=True)

    def _pair(k, carry):
        c0 = 2 * k
        _stage_idx(c0 + 1, gcur1, dcur1)
        _gather(gcur1, rows1, sem1)
        _drain(gcur0, rows0, sem0)
        _scatter(dcur0, rows0)
        _stage_idx(c0 + 2, gcur0, dcur0)
        _gather(gcur0, rows0, sem0)
        _drain(gcur1, rows1, sem1)
        _scatter(dcur1, rows1)
        return carry

    for base, n in PASS_SPLITS:
        pltpu.sync_copy(g_hbm.at[wid, pl.ds(base, n)], gbuf.at[pl.ds(0, n)])
        pltpu.sync_copy(d_hbm.at[wid, pl.ds(base, n)], dbuf.at[pl.ds(0, n)])
        _stage_idx(0, gcur0, dcur0)
        _gather(gcur0, rows0, sem0)
        if n % 2 == 0:
            lax.fori_loop(0, n // 2 - 1, _pair, 0)
            _stage_idx(n - 1, gcur1, dcur1)
            _gather(gcur1, rows1, sem1)
            _drain(gcur0, rows0, sem0)
            _scatter(dcur0, rows0)
            _drain(gcur1, rows1, sem1)
            _scatter(dcur1, rows1)
        else:
            lax.fori_loop(0, (n - 1) // 2, _pair, 0)
            _drain(gcur0, rows0, sem0)
            _scatter(dcur0, rows0)

    plsc.subcore_barrier()

    for k in range(SR // DC):
        r0 = sid * SR + k * DC
        pltpu.sync_copy(agg.at[pl.ds(r0, DC)], zbuf)
        pltpu.sync_copy(zbuf, out_hbm.at[pl.ds(cid * NP + r0, DC)])


def _sc_agg(g3, d3, xall_flat):
    mesh = plsc.VectorSubcoreMesh(core_axis_name="c", subcore_axis_name="s")
    kern = pl.kernel(
        _sc_body,
        mesh=mesh,
        out_type=jax.ShapeDtypeStruct((NC * NP, D), jnp.float32),
        scratch_types=[
            pltpu.VMEM((PB, CH), jnp.int32),
            pltpu.VMEM((PB, CH), jnp.int32),
            pltpu.VMEM((CH,), jnp.int32),
            pltpu.VMEM((CH,), jnp.int32),
            pltpu.VMEM((CH,), jnp.int32),
            pltpu.VMEM((CH,), jnp.int32),
            pltpu.VMEM((CH, D), jnp.float32),
            pltpu.VMEM((CH, D), jnp.float32),
            pltpu.VMEM((DC, D), jnp.float32),
            pltpu.VMEM_SHARED((NP, D), jnp.float32),
            pltpu.SemaphoreType.DMA,
            pltpu.SemaphoreType.DMA,
        ],
    )
    return kern(g3, d3, xall_flat)


BO = 80


def _out2_body(a0_ref, a1_ref, x_ref, root_ref, bias_ref, ir_ref, wr_ref,
               o_ref, rel_ref):
    @pl.when(pl.program_id(0) == 0)
    def _():
        rel_ref[...] = jnp.dot(ir_ref[...], wr_ref[...],
                               preferred_element_type=jnp.float32)

    o_ref[...] = (a0_ref[...] + a1_ref[...] + bias_ref[...]
                  + jnp.dot(x_ref[...], root_ref[...],
                            preferred_element_type=jnp.float32))


def _out2(agg2, x, root, bias, init_rel, w_rel):
    return pl.pallas_call(
        _out2_body,
        grid=(N // BO,),
        in_specs=[
            pl.BlockSpec((BO, D), lambda i: (i, 0)),
            pl.BlockSpec((BO, D), lambda i: (i + NP // BO, 0)),
            pl.BlockSpec((BO, D), lambda i: (i, 0)),
            pl.BlockSpec((D, D), lambda i: (0, 0)),
            pl.BlockSpec((1, D), lambda i: (0, 0)),
            pl.BlockSpec((2 * R, D), lambda i: (0, 0)),
            pl.BlockSpec((D, D), lambda i: (0, 0)),
        ],
        out_specs=[
            pl.BlockSpec((BO, D), lambda i: (i, 0)),
            pl.BlockSpec((2 * R, D), lambda i: (0, 0)),
        ],
        out_shape=[
            jax.ShapeDtypeStruct((N, D), jnp.float32),
            jax.ShapeDtypeStruct((2 * R, D), jnp.float32),
        ],
    )(agg2, agg2, x, root, bias.reshape(1, D), init_rel, w_rel)


def kernel(edge_index, edge_type, init_embed, init_rel, w_rel,
           basis1, comp1, root1, bias1, basis2, comp2, root2, bias2):
    def _pad_edges(a, val):
        a2 = a.reshape(NW, EW)
        a2 = jnp.pad(a2, ((0, 0), (0, EWP - EW)), constant_values=val)
        return a2.reshape(NW * NCH, CH)

    t2 = _pad_edges(edge_type, 0)
    s2 = _pad_edges(edge_index[0], 0)
    d3 = _pad_edges(edge_index[1], N).reshape(NW, NCH, CH)

    xall1, g2 = _xall1(init_embed, basis1, comp1, t2, s2)
    g3 = g2.reshape(NW, NCH, CH)
    agg1 = _sc_agg(g3, d3, xall1.reshape(R * N, D))
    xall2, x1 = _xall2(agg1, init_embed, root1, bias1, basis2, comp2)
    agg2 = _sc_agg(g3, d3, xall2.reshape(R * N, D))
    x2, rel = _out2(agg2, x1, root2, bias2, init_rel, w_rel)
    return (x2, rel)


# speedup vs baseline: 1.7585x; 1.7585x over previous
import functools

import jax
import jax.numpy as jnp
from jax import lax
from jax.experimental import pallas as pl
from jax.experimental.pallas import tpu as pltpu
from jax.experimental.pallas import tpu_sc as plsc

N = 10000
E = 320000
D = 128
R = 32
NB = 8

NC = 2
NS = 16
NW = NC * NS
EW = E // NW
CH = 80
NCH = EW // CH
LANES = 16
NP = 10240
SR = NP // NS
DC = 40


def _wr(comp_smem, basis_ref, r):
    acc = comp_smem[r, 0] * basis_ref[0]
    for b in range(1, NB):
        acc = acc + comp_smem[r, b] * basis_ref[b]
    return acc


def _xall1_body(comp_smem, x_ref, basis_ref, t_ref, s_ref, out_ref, g_ref):
    r = pl.program_id(0)

    @pl.when(r == 0)
    def _():
        g_ref[...] = t_ref[...] * N + s_ref[...]

    out_ref[0] = jnp.dot(x_ref[...], _wr(comp_smem, basis_ref, r),
                         preferred_element_type=jnp.float32)


def _xall1(x, basis, comp, t2, s2):
    return pl.pallas_call(
        _xall1_body,
        grid=(R,),
        in_specs=[
            pl.BlockSpec(memory_space=pltpu.SMEM),
            pl.BlockSpec((N, D), lambda r: (0, 0)),
            pl.BlockSpec((NB, D, D), lambda r: (0, 0, 0)),
            pl.BlockSpec((NW * NCH, CH), lambda r: (0, 0)),
            pl.BlockSpec((NW * NCH, CH), lambda r: (0, 0)),
        ],
        out_specs=[
            pl.BlockSpec((1, N, D), lambda r: (r, 0, 0)),
            pl.BlockSpec((NW * NCH, CH), lambda r: (0, 0)),
        ],
        out_shape=[
            jax.ShapeDtypeStruct((R, N, D), jnp.float32),
            jax.ShapeDtypeStruct((NW * NCH, CH), jnp.int32),
        ],
    )(comp, x, basis, t2, s2)


def _xall2_body(comp_smem, agga_ref, aggb_ref, x0_ref, root_ref, bias_ref,
                basis_ref, out_ref, x1_ref, x1s_ref):
    r = pl.program_id(0)

    @pl.when(r == 0)
    def _():
        x1 = jnp.tanh(agga_ref[0:N] + aggb_ref[0:N] + bias_ref[...]
                      + jnp.dot(x0_ref[...], root_ref[...],
                                preferred_element_type=jnp.float32))
        x1s_ref[...] = x1
        x1_ref[...] = x1

    out_ref[0] = jnp.dot(x1s_ref[...], _wr(comp_smem, basis_ref, r),
                         preferred_element_type=jnp.float32)


def _xall2(agg1, x0, root1, bias1, basis, comp):
    return pl.pallas_call(
        _xall2_body,
        grid=(R,),
        in_specs=[
            pl.BlockSpec(memory_space=pltpu.SMEM),
            pl.BlockSpec((NP, D), lambda r: (0, 0)),
            pl.BlockSpec((NP, D), lambda r: (1, 0)),
            pl.BlockSpec((N, D), lambda r: (0, 0)),
            pl.BlockSpec((D, D), lambda r: (0, 0)),
            pl.BlockSpec((1, D), lambda r: (0, 0)),
            pl.BlockSpec((NB, D, D), lambda r: (0, 0, 0)),
        ],
        out_specs=[
            pl.BlockSpec((1, N, D), lambda r: (r, 0, 0)),
            pl.BlockSpec((N, D), lambda r: (0, 0)),
        ],
        out_shape=[
            jax.ShapeDtypeStruct((R, N, D), jnp.float32),
            jax.ShapeDtypeStruct((N, D), jnp.float32),
        ],
        scratch_shapes=[pltpu.VMEM((N, D), jnp.float32)],
    )(comp, agg1, agg1, x0, root1, bias1.reshape(1, D), basis)


PASS_SPLITS = ((0, 64), (64, 61))
PB = 64


def _sc_body(g_hbm, d_hbm, xall_hbm, out_hbm,
             gbuf, dbuf, gcur0, gcur1, dcur0, dcur1, rows0, rows1,
             zbuf, agg, sem0, sem1):
    cid = lax.axis_index("c")
    sid = lax.axis_index("s")
    wid = cid * NS + sid

    def _zr(i, carry):
        for j in range(D // LANES):
            zbuf[i, pl.ds(j * LANES, LANES)] = jnp.zeros((LANES,), jnp.float32)
        return carry

    lax.fori_loop(0, DC, _zr, 0)
    for k in range(SR // DC):
        pltpu.sync_copy(zbuf, agg.at[pl.ds(sid * SR + k * DC, DC)])
    plsc.subcore_barrier()

    def _stage_idx(c, gcur, dcur):
        for j in range(CH // LANES):
            sl = pl.ds(j * LANES, LANES)
            gcur[sl] = gbuf[c, sl]
            dcur[sl] = dbuf[c, sl]

    def _gather(gcur, buf, sem):
        pltpu.make_async_copy(xall_hbm.at[gcur], buf, sem).start()

    def _drain(gcur, buf, sem):
        pltpu.make_async_copy(xall_hbm.at[gcur], buf, sem).wait()

    def _scatter(dcur, buf):
        pltpu.sync_copy(buf, agg.at[dcur], add=True)

    def _pair(k, carry):
        c0 = 2 * k
        _stage_idx(c0 + 1, gcur1, dcur1)
        _gather(gcur1, rows1, sem1)
        _drain(gcur0, rows0, sem0)
        _scatter(dcur0, rows0)
        _stage_idx(c0 + 2, gcur0, dcur0)
        _gather(gcur0, rows0, sem0)
        _drain(gcur1, rows1, sem1)
        _scatter(dcur1, rows1)
        return carry

    for base, n in PASS_SPLITS:
        pltpu.sync_copy(g_hbm.at[wid, pl.ds(base, n)], gbuf.at[pl.ds(0, n)])
        pltpu.sync_copy(d_hbm.at[wid, pl.ds(base, n)], dbuf.at[pl.ds(0, n)])
        _stage_idx(0, gcur0, dcur0)
        _gather(gcur0, rows0, sem0)
        if n % 2 == 0:
            lax.fori_loop(0, n // 2 - 1, _pair, 0)
            _stage_idx(n - 1, gcur1, dcur1)
            _gather(gcur1, rows1, sem1)
            _drain(gcur0, rows0, sem0)
            _scatter(dcur0, rows0)
            _drain(gcur1, rows1, sem1)
            _scatter(dcur1, rows1)
        else:
            lax.fori_loop(0, (n - 1) // 2, _pair, 0)
            _drain(gcur0, rows0, sem0)
            _scatter(dcur0, rows0)

    plsc.subcore_barrier()

    for k in range(SR // DC):
        r0 = sid * SR + k * DC
        pltpu.sync_copy(agg.at[pl.ds(r0, DC)], zbuf)
        pltpu.sync_copy(zbuf, out_hbm.at[pl.ds(cid * NP + r0, DC)])


def _sc_agg(g3, d3, xall_flat):
    mesh = plsc.VectorSubcoreMesh(core_axis_name="c", subcore_axis_name="s")
    kern = pl.kernel(
        _sc_body,
        mesh=mesh,
        out_type=jax.ShapeDtypeStruct((NC * NP, D), jnp.float32),
        scratch_types=[
            pltpu.VMEM((PB, CH), jnp.int32),
            pltpu.VMEM((PB, CH), jnp.int32),
            pltpu.VMEM((CH,), jnp.int32),
            pltpu.VMEM((CH,), jnp.int32),
            pltpu.VMEM((CH,), jnp.int32),
            pltpu.VMEM((CH,), jnp.int32),
            pltpu.VMEM((CH, D), jnp.float32),
            pltpu.VMEM((CH, D), jnp.float32),
            pltpu.VMEM((DC, D), jnp.float32),
            pltpu.VMEM_SHARED((NP, D), jnp.float32),
            pltpu.SemaphoreType.DMA,
            pltpu.SemaphoreType.DMA,
        ],
    )
    return kern(g3, d3, xall_flat)


BO = 80


def _out2_body(a0_ref, a1_ref, x_ref, root_ref, bias_ref, ir_ref, wr_ref,
               o_ref, rel_ref):
    rel_ref[...] = jnp.dot(ir_ref[...], wr_ref[...],
                           preferred_element_type=jnp.float32)
    o_ref[...] = (a0_ref[0:N] + a1_ref[0:N] + bias_ref[...]
                  + jnp.dot(x_ref[...], root_ref[...],
                            preferred_element_type=jnp.float32))


def _out2(agg2, x, root, bias, init_rel, w_rel):
    return pl.pallas_call(
        _out2_body,
        grid=(1,),
        in_specs=[
            pl.BlockSpec((NP, D), lambda i: (0, 0)),
            pl.BlockSpec((NP, D), lambda i: (1, 0)),
            pl.BlockSpec((N, D), lambda i: (0, 0)),
            pl.BlockSpec((D, D), lambda i: (0, 0)),
            pl.BlockSpec((1, D), lambda i: (0, 0)),
            pl.BlockSpec((2 * R, D), lambda i: (0, 0)),
            pl.BlockSpec((D, D), lambda i: (0, 0)),
        ],
        out_specs=[
            pl.BlockSpec((N, D), lambda i: (0, 0)),
            pl.BlockSpec((2 * R, D), lambda i: (0, 0)),
        ],
        out_shape=[
            jax.ShapeDtypeStruct((N, D), jnp.float32),
            jax.ShapeDtypeStruct((2 * R, D), jnp.float32),
        ],
    )(agg2, agg2, x, root, bias.reshape(1, D), init_rel, w_rel)


def kernel(edge_index, edge_type, init_embed, init_rel, w_rel,
           basis1, comp1, root1, bias1, basis2, comp2, root2, bias2):
    t2 = edge_type.reshape(NW * NCH, CH)
    s2 = edge_index[0].reshape(NW * NCH, CH)
    d3 = edge_index[1].reshape(NW, NCH, CH)

    xall1, g2 = _xall1(init_embed, basis1, comp1, t2, s2)
    g3 = g2.reshape(NW, NCH, CH)
    agg1 = _sc_agg(g3, d3, xall1.reshape(R * N, D))
    xall2, x1 = _xall2(agg1, init_embed, root1, bias1, basis2, comp2)
    agg2 = _sc_agg(g3, d3, xall2.reshape(R * N, D))
    x2, rel = _out2(agg2, x1, root2, bias2, init_rel, w_rel)
    return (x2, rel)


# speedup vs baseline: 1.7782x; 1.0112x over previous
import functools

import jax
import jax.numpy as jnp
from jax import lax
from jax.experimental import pallas as pl
from jax.experimental.pallas import tpu as pltpu
from jax.experimental.pallas import tpu_sc as plsc

N = 10000
E = 320000
D = 128
R = 32
NB = 8

NC = 2
NS = 16
NW = NC * NS
EW = E // NW
CH = 80
NCH = EW // CH
LANES = 16
NP = 10240
SR = NP // NS
DC = 80


def _wr(comp_smem, basis_ref, r):
    acc = comp_smem[r, 0] * basis_ref[0]
    for b in range(1, NB):
        acc = acc + comp_smem[r, b] * basis_ref[b]
    return acc


def _xall1_body(comp_smem, x_ref, basis_ref, t_ref, s_ref, out_ref, g_ref):
    r = pl.program_id(0)

    @pl.when(r == 0)
    def _():
        g_ref[...] = t_ref[...] * N + s_ref[...]

    out_ref[0] = jnp.dot(x_ref[...], _wr(comp_smem, basis_ref, r),
                         preferred_element_type=jnp.float32)


def _xall1(x, basis, comp, t2, s2):
    return pl.pallas_call(
        _xall1_body,
        grid=(R,),
        in_specs=[
            pl.BlockSpec(memory_space=pltpu.SMEM),
            pl.BlockSpec((N, D), lambda r: (0, 0)),
            pl.BlockSpec((NB, D, D), lambda r: (0, 0, 0)),
            pl.BlockSpec((NW * NCH, CH), lambda r: (0, 0)),
            pl.BlockSpec((NW * NCH, CH), lambda r: (0, 0)),
        ],
        out_specs=[
            pl.BlockSpec((1, N, D), lambda r: (r, 0, 0)),
            pl.BlockSpec((NW * NCH, CH), lambda r: (0, 0)),
        ],
        out_shape=[
            jax.ShapeDtypeStruct((R, N, D), jnp.float32),
            jax.ShapeDtypeStruct((NW * NCH, CH), jnp.int32),
        ],
    )(comp, x, basis, t2, s2)


def _xall2_body(comp_smem, agga_ref, aggb_ref, x0_ref, root_ref, bias_ref,
                basis_ref, out_ref, x1_ref, x1s_ref):
    r = pl.program_id(0)

    @pl.when(r == 0)
    def _():
        x1 = jnp.tanh(agga_ref[0:N] + aggb_ref[0:N] + bias_ref[...]
                      + jnp.dot(x0_ref[...], root_ref[...],
                                preferred_element_type=jnp.float32))
        x1s_ref[...] = x1
        x1_ref[...] = x1

    out_ref[0] = jnp.dot(x1s_ref[...], _wr(comp_smem, basis_ref, r),
                         preferred_element_type=jnp.float32)


def _xall2(agg1, x0, root1, bias1, basis, comp):
    return pl.pallas_call(
        _xall2_body,
        grid=(R,),
        in_specs=[
            pl.BlockSpec(memory_space=pltpu.SMEM),
            pl.BlockSpec((NP, D), lambda r: (0, 0)),
            pl.BlockSpec((NP, D), lambda r: (1, 0)),
            pl.BlockSpec((N, D), lambda r: (0, 0)),
            pl.BlockSpec((D, D), lambda r: (0, 0)),
            pl.BlockSpec((1, D), lambda r: (0, 0)),
            pl.BlockSpec((NB, D, D), lambda r: (0, 0, 0)),
        ],
        out_specs=[
            pl.BlockSpec((1, N, D), lambda r: (r, 0, 0)),
            pl.BlockSpec((N, D), lambda r: (0, 0)),
        ],
        out_shape=[
            jax.ShapeDtypeStruct((R, N, D), jnp.float32),
            jax.ShapeDtypeStruct((N, D), jnp.float32),
        ],
        scratch_shapes=[pltpu.VMEM((N, D), jnp.float32)],
    )(comp, agg1, agg1, x0, root1, bias1.reshape(1, D), basis)


PASS_SPLITS = ((0, 64), (64, 61))
PB = 64


def _sc_body(g_hbm, d_hbm, xall_hbm, out_hbm,
             gbuf, dbuf, gcur0, gcur1, dcur0, dcur1, rows0, rows1,
             zbuf, agg, sem0, sem1):
    cid = lax.axis_index("c")
    sid = lax.axis_index("s")
    wid = cid * NS + sid

    def _zr(i, carry):
        for j in range(D // LANES):
            zbuf[i, pl.ds(j * LANES, LANES)] = jnp.zeros((LANES,), jnp.float32)
        return carry

    lax.fori_loop(0, DC, _zr, 0)
    for k in range(SR // DC):
        pltpu.sync_copy(zbuf, agg.at[pl.ds(sid * SR + k * DC, DC)])
    plsc.subcore_barrier()

    def _stage_idx(c, gcur, dcur):
        for j in range(CH // LANES):
            sl = pl.ds(j * LANES, LANES)
            gcur[sl] = gbuf[c, sl]
            dcur[sl] = dbuf[c, sl]

    def _gather(gcur, buf, sem):
        pltpu.make_async_copy(xall_hbm.at[gcur], buf, sem).start()

    def _drain(gcur, buf, sem):
        pltpu.make_async_copy(xall_hbm.at[gcur], buf, sem).wait()

    def _scatter(dcur, buf):
        pltpu.sync_copy(buf, agg.at[dcur], add=True)

    def _pair(k, carry):
        c0 = 2 * k
        _stage_idx(c0 + 1, gcur1, dcur1)
        _gather(gcur1, rows1, sem1)
        _drain(gcur0, rows0, sem0)
        _scatter(dcur0, rows0)
        _stage_idx(c0 + 2, gcur0, dcur0)
        _gather(gcur0, rows0, sem0)
        _drain(gcur1, rows1, sem1)
        _scatter(dcur1, rows1)
        return carry

    for base, n in PASS_SPLITS:
        pltpu.sync_copy(g_hbm.at[wid, pl.ds(base, n)], gbuf.at[pl.ds(0, n)])
        pltpu.sync_copy(d_hbm.at[wid, pl.ds(base, n)], dbuf.at[pl.ds(0, n)])
        _stage_idx(0, gcur0, dcur0)
        _gather(gcur0, rows0, sem0)
        if n % 2 == 0:
            lax.fori_loop(0, n // 2 - 1, _pair, 0)
            _stage_idx(n - 1, gcur1, dcur1)
            _gather(gcur1, rows1, sem1)
            _drain(gcur0, rows0, sem0)
            _scatter(dcur0, rows0)
            _drain(gcur1, rows1, sem1)
            _scatter(dcur1, rows1)
        else:
            lax.fori_loop(0, (n - 1) // 2, _pair, 0)
            _drain(gcur0, rows0, sem0)
            _scatter(dcur0, rows0)

    plsc.subcore_barrier()

    r0 = sid * SR
    pltpu.sync_copy(agg.at[pl.ds(r0, SR)], out_hbm.at[pl.ds(cid * NP + r0, SR)])


def _sc_agg(g3, d3, xall_flat):
    mesh = plsc.VectorSubcoreMesh(core_axis_name="c", subcore_axis_name="s")
    kern = pl.kernel(
        _sc_body,
        mesh=mesh,
        out_type=jax.ShapeDtypeStruct((NC * NP, D), jnp.float32),
        scratch_types=[
            pltpu.VMEM((PB, CH), jnp.int32),
            pltpu.VMEM((PB, CH), jnp.int32),
            pltpu.VMEM((CH,), jnp.int32),
            pltpu.VMEM((CH,), jnp.int32),
            pltpu.VMEM((CH,), jnp.int32),
            pltpu.VMEM((CH,), jnp.int32),
            pltpu.VMEM((CH, D), jnp.float32),
            pltpu.VMEM((CH, D), jnp.float32),
            pltpu.VMEM((DC, D), jnp.float32),
            pltpu.VMEM_SHARED((NP, D), jnp.float32),
            pltpu.SemaphoreType.DMA,
            pltpu.SemaphoreType.DMA,
        ],
    )
    return kern(g3, d3, xall_flat)


BO = 80


def _out2_body(a0_ref, a1_ref, x_ref, root_ref, bias_ref, ir_ref, wr_ref,
               o_ref, rel_ref):
    rel_ref[...] = jnp.dot(ir_ref[...], wr_ref[...],
                           preferred_element_type=jnp.float32)
    o_ref[...] = (a0_ref[0:N] + a1_ref[0:N] + bias_ref[...]
                  + jnp.dot(x_ref[...], root_ref[...],
                            preferred_element_type=jnp.float32))


def _out2(agg2, x, root, bias, init_rel, w_rel):
    return pl.pallas_call(
        _out2_body,
        grid=(1,),
        in_specs=[
            pl.BlockSpec((NP, D), lambda i: (0, 0)),
            pl.BlockSpec((NP, D), lambda i: (1, 0)),
            pl.BlockSpec((N, D), lambda i: (0, 0)),
            pl.BlockSpec((D, D), lambda i: (0, 0)),
            pl.BlockSpec((1, D), lambda i: (0, 0)),
            pl.BlockSpec((2 * R, D), lambda i: (0, 0)),
            pl.BlockSpec((D, D), lambda i: (0, 0)),
        ],
        out_specs=[
            pl.BlockSpec((N, D), lambda i: (0, 0)),
            pl.BlockSpec((2 * R, D), lambda i: (0, 0)),
        ],
        out_shape=[
            jax.ShapeDtypeStruct((N, D), jnp.float32),
            jax.ShapeDtypeStruct((2 * R, D), jnp.float32),
        ],
    )(agg2, agg2, x, root, bias.reshape(1, D), init_rel, w_rel)


def kernel(edge_index, edge_type, init_embed, init_rel, w_rel,
           basis1, comp1, root1, bias1, basis2, comp2, root2, bias2):
    t2 = edge_type.reshape(NW * NCH, CH)
    s2 = edge_index[0].reshape(NW * NCH, CH)
    d3 = edge_index[1].reshape(NW, NCH, CH)

    xall1, g2 = _xall1(init_embed, basis1, comp1, t2, s2)
    g3 = g2.reshape(NW, NCH, CH)
    agg1 = _sc_agg(g3, d3, xall1.reshape(R * N, D))
    xall2, x1 = _xall2(agg1, init_embed, root1, bias1, basis2, comp2)
    agg2 = _sc_agg(g3, d3, xall2.reshape(R * N, D))
    x2, rel = _out2(agg2, x1, root2, bias2, init_rel, w_rel)
    return (x2, rel)


# speedup vs baseline: 1.7815x; 1.0019x over previous
import functools

import jax
import jax.numpy as jnp
from jax import lax
from jax.experimental import pallas as pl
from jax.experimental.pallas import tpu as pltpu
from jax.experimental.pallas import tpu_sc as plsc

N = 10000
E = 320000
D = 128
R = 32
NB = 8

NC = 2
NS = 16
NW = NC * NS
EW = E // NW
CH = 80
NCH = EW // CH
LANES = 16
NP = 10240
SR = NP // NS
DC = 80


def _wr(comp_smem, basis_ref, r):
    acc = comp_smem[r, 0] * basis_ref[0]
    for b in range(1, NB):
        acc = acc + comp_smem[r, b] * basis_ref[b]
    return acc


def _xall1_body(comp_smem, x_ref, basis_ref, t_ref, s_ref, out_ref, g_ref):
    r = pl.program_id(0)

    @pl.when(r == 0)
    def _():
        g_ref[...] = t_ref[...] * N + s_ref[...]

    out_ref[0] = jnp.dot(x_ref[...], _wr(comp_smem, basis_ref, r),
                         preferred_element_type=jnp.float32)


def _xall1(x, basis, comp, t2, s2):
    return pl.pallas_call(
        _xall1_body,
        grid=(R,),
        in_specs=[
            pl.BlockSpec(memory_space=pltpu.SMEM),
            pl.BlockSpec((N, D), lambda r: (0, 0)),
            pl.BlockSpec((NB, D, D), lambda r: (0, 0, 0)),
            pl.BlockSpec((NW * NCH, CH), lambda r: (0, 0)),
            pl.BlockSpec((NW * NCH, CH), lambda r: (0, 0)),
        ],
        out_specs=[
            pl.BlockSpec((1, N, D), lambda r: (r, 0, 0)),
            pl.BlockSpec((NW * NCH, CH), lambda r: (0, 0)),
        ],
        out_shape=[
            jax.ShapeDtypeStruct((R, N, D), jnp.float32),
            jax.ShapeDtypeStruct((NW * NCH, CH), jnp.int32),
        ],
    )(comp, x, basis, t2, s2)


def _xall2_body(comp_smem, agga_ref, aggb_ref, x0_ref, root_ref, bias_ref,
                basis_ref, out_ref, x1_ref, x1s_ref):
    r = pl.program_id(0)

    @pl.when(r == 0)
    def _():
        x1 = jnp.tanh(agga_ref[0:N] + aggb_ref[0:N] + bias_ref[...]
                      + jnp.dot(x0_ref[...], root_ref[...],
                                preferred_element_type=jnp.float32))
        x1s_ref[...] = x1
        x1_ref[...] = x1

    out_ref[0] = jnp.dot(x1s_ref[...], _wr(comp_smem, basis_ref, r),
                         preferred_element_type=jnp.float32)


def _xall2(agg1, x0, root1, bias1, basis, comp):
    return pl.pallas_call(
        _xall2_body,
        grid=(R,),
        in_specs=[
            pl.BlockSpec(memory_space=pltpu.SMEM),
            pl.BlockSpec((NP, D), lambda r: (0, 0)),
            pl.BlockSpec((NP, D), lambda r: (1, 0)),
            pl.BlockSpec((N, D), lambda r: (0, 0)),
            pl.BlockSpec((D, D), lambda r: (0, 0)),
            pl.BlockSpec((1, D), lambda r: (0, 0)),
            pl.BlockSpec((NB, D, D), lambda r: (0, 0, 0)),
        ],
        out_specs=[
            pl.BlockSpec((1, N, D), lambda r: (r, 0, 0)),
            pl.BlockSpec((N, D), lambda r: (0, 0)),
        ],
        out_shape=[
            jax.ShapeDtypeStruct((R, N, D), jnp.float32),
            jax.ShapeDtypeStruct((N, D), jnp.float32),
        ],
        scratch_shapes=[pltpu.VMEM((N, D), jnp.float32)],
    )(comp, agg1, agg1, x0, root1, bias1.reshape(1, D), basis)


PASS_SPLITS = ((0, 64), (64, 61))
PB = 64


def _sc_body(g_hbm, d_hbm, xall_hbm, out_hbm,
             gbuf, dbuf, rows0, rows1, zbuf, agg, sem0, sem1):
    cid = lax.axis_index("c")
    sid = lax.axis_index("s")
    wid = cid * NS + sid

    def _zr(i, carry):
        for j in range(D // LANES):
            zbuf[i, pl.ds(j * LANES, LANES)] = jnp.zeros((LANES,), jnp.float32)
        return carry

    lax.fori_loop(0, DC, _zr, 0)
    for k in range(SR // DC):
        pltpu.sync_copy(zbuf, agg.at[pl.ds(sid * SR + k * DC, DC)])
    plsc.subcore_barrier()

    rows = (rows0, rows1)
    sems = (sem0, sem1)

    def _gather(c, p):
        pltpu.make_async_copy(xall_hbm.at[gbuf.at[c]], rows[p], sems[p]).start()

    def _drain(c, p):
        pltpu.make_async_copy(xall_hbm.at[gbuf.at[c]], rows[p], sems[p]).wait()

    def _scatter(c, p):
        pltpu.sync_copy(rows[p], agg.at[dbuf.at[c]], add=True)

    for base, n in PASS_SPLITS:
        pltpu.sync_copy(g_hbm.at[wid, pl.ds(base, n)], gbuf.at[pl.ds(0, n)])
        pltpu.sync_copy(d_hbm.at[wid, pl.ds(base, n)], dbuf.at[pl.ds(0, n)])
        _gather(0, 0)
        for c in range(1, n):
            _gather(c, c % 2)
            _drain(c - 1, (c - 1) % 2)
            _scatter(c - 1, (c - 1) % 2)
        _drain(n - 1, (n - 1) % 2)
        _scatter(n - 1, (n - 1) % 2)

    plsc.subcore_barrier()

    r0 = sid * SR
    pltpu.sync_copy(agg.at[pl.ds(r0, SR)], out_hbm.at[pl.ds(cid * NP + r0, SR)])


def _sc_agg(g3, d3, xall_flat):
    mesh = plsc.VectorSubcoreMesh(core_axis_name="c", subcore_axis_name="s")
    kern = pl.kernel(
        _sc_body,
        mesh=mesh,
        out_type=jax.ShapeDtypeStruct((NC * NP, D), jnp.float32),
        scratch_types=[
            pltpu.VMEM((PB, CH), jnp.int32),
            pltpu.VMEM((PB, CH), jnp.int32),
            pltpu.VMEM((CH, D), jnp.float32),
            pltpu.VMEM((CH, D), jnp.float32),
            pltpu.VMEM((DC, D), jnp.float32),
            pltpu.VMEM_SHARED((NP, D), jnp.float32),
            pltpu.SemaphoreType.DMA,
            pltpu.SemaphoreType.DMA,
        ],
    )
    return kern(g3, d3, xall_flat)


BO = 80


def _out2_body(a0_ref, a1_ref, x_ref, root_ref, bias_ref, ir_ref, wr_ref,
               o_ref, rel_ref):
    rel_ref[...] = jnp.dot(ir_ref[...], wr_ref[...],
                           preferred_element_type=jnp.float32)
    o_ref[...] = (a0_ref[0:N] + a1_ref[0:N] + bias_ref[...]
                  + jnp.dot(x_ref[...], root_ref[...],
                            preferred_element_type=jnp.float32))


def _out2(agg2, x, root, bias, init_rel, w_rel):
    return pl.pallas_call(
        _out2_body,
        grid=(1,),
        in_specs=[
            pl.BlockSpec((NP, D), lambda i: (0, 0)),
            pl.BlockSpec((NP, D), lambda i: (1, 0)),
            pl.BlockSpec((N, D), lambda i: (0, 0)),
            pl.BlockSpec((D, D), lambda i: (0, 0)),
            pl.BlockSpec((1, D), lambda i: (0, 0)),
            pl.BlockSpec((2 * R, D), lambda i: (0, 0)),
            pl.BlockSpec((D, D), lambda i: (0, 0)),
        ],
        out_specs=[
            pl.BlockSpec((N, D), lambda i: (0, 0)),
            pl.BlockSpec((2 * R, D), lambda i: (0, 0)),
        ],
        out_shape=[
            jax.ShapeDtypeStruct((N, D), jnp.float32),
            jax.ShapeDtypeStruct((2 * R, D), jnp.float32),
        ],
    )(agg2, agg2, x, root, bias.reshape(1, D), init_rel, w_rel)


def kernel(edge_index, edge_type, init_embed, init_rel, w_rel,
           basis1, comp1, root1, bias1, basis2, comp2, root2, bias2):
    t2 = edge_type.reshape(NW * NCH, CH)
    s2 = edge_index[0].reshape(NW * NCH, CH)
    d3 = edge_index[1].reshape(NW, NCH, CH)

    xall1, g2 = _xall1(init_embed, basis1, comp1, t2, s2)
    g3 = g2.reshape(NW, NCH, CH)
    agg1 = _sc_agg(g3, d3, xall1.reshape(R * N, D))
    xall2, x1 = _xall2(agg1, init_embed, root1, bias1, basis2, comp2)
    agg2 = _sc_agg(g3, d3, xall2.reshape(R * N, D))
    x2, rel = _out2(agg2, x1, root2, bias2, init_rel, w_rel)
    return (x2, rel)


# speedup vs baseline: 1.8101x; 1.0161x over previous
import functools

import jax
import jax.numpy as jnp
from jax import lax
from jax.experimental import pallas as pl
from jax.experimental.pallas import tpu as pltpu
from jax.experimental.pallas import tpu_sc as plsc

N = 10000
E = 320000
D = 128
R = 32
NB = 8

NC = 2
NS = 16
NW = NC * NS
EW = E // NW
CH = 80
NCH = EW // CH
LANES = 16
NP = 10240
SR = NP // NS
DC = 80


def _wr(comp_smem, basis_ref, r):
    acc = comp_smem[r, 0] * basis_ref[0]
    for b in range(1, NB):
        acc = acc + comp_smem[r, b] * basis_ref[b]
    return acc


def _xall1_body(comp_smem, x_ref, basis_ref, t_ref, ei_ref, out_ref, g_ref):
    r = pl.program_id(0)

    @pl.when(r == 0)
    def _():
        g_ref[...] = t_ref[...] * N + ei_ref[0]

    out_ref[0] = jnp.dot(x_ref[...], _wr(comp_smem, basis_ref, r),
                         preferred_element_type=jnp.float32)


def _xall1(x, basis, comp, t2, ei2):
    return pl.pallas_call(
        _xall1_body,
        grid=(R,),
        in_specs=[
            pl.BlockSpec(memory_space=pltpu.SMEM),
            pl.BlockSpec((N, D), lambda r: (0, 0)),
            pl.BlockSpec((NB, D, D), lambda r: (0, 0, 0)),
            pl.BlockSpec((NW * NCH, CH), lambda r: (0, 0)),
            pl.BlockSpec((2, NW * NCH, CH), lambda r: (0, 0, 0)),
        ],
        out_specs=[
            pl.BlockSpec((1, N, D), lambda r: (r, 0, 0)),
            pl.BlockSpec((NW * NCH, CH), lambda r: (0, 0)),
        ],
        out_shape=[
            jax.ShapeDtypeStruct((R, N, D), jnp.float32),
            jax.ShapeDtypeStruct((NW * NCH, CH), jnp.int32),
        ],
    )(comp, x, basis, t2, ei2)


def _xall2_body(comp_smem, agga_ref, aggb_ref, x0_ref, root_ref, bias_ref,
                basis_ref, out_ref, x1_ref, x1s_ref):
    r = pl.program_id(0)

    @pl.when(r == 0)
    def _():
        x1 = jnp.tanh(agga_ref[0:N] + aggb_ref[0:N] + bias_ref[...]
                      + jnp.dot(x0_ref[...], root_ref[...],
                                preferred_element_type=jnp.float32))
        x1s_ref[...] = x1
        x1_ref[...] = x1

    out_ref[0] = jnp.dot(x1s_ref[...], _wr(comp_smem, basis_ref, r),
                         preferred_element_type=jnp.float32)


def _xall2(agg1, x0, root1, bias1, basis, comp):
    return pl.pallas_call(
        _xall2_body,
        grid=(R,),
        in_specs=[
            pl.BlockSpec(memory_space=pltpu.SMEM),
            pl.BlockSpec((NP, D), lambda r: (0, 0)),
            pl.BlockSpec((NP, D), lambda r: (1, 0)),
            pl.BlockSpec((N, D), lambda r: (0, 0)),
            pl.BlockSpec((D, D), lambda r: (0, 0)),
            pl.BlockSpec((1, D), lambda r: (0, 0)),
            pl.BlockSpec((NB, D, D), lambda r: (0, 0, 0)),
        ],
        out_specs=[
            pl.BlockSpec((1, N, D), lambda r: (r, 0, 0)),
            pl.BlockSpec((N, D), lambda r: (0, 0)),
        ],
        out_shape=[
            jax.ShapeDtypeStruct((R, N, D), jnp.float32),
            jax.ShapeDtypeStruct((N, D), jnp.float32),
        ],
        scratch_shapes=[pltpu.VMEM((N, D), jnp.float32)],
    )(comp, agg1, agg1, x0, root1, bias1.reshape(1, D), basis)


PASS_SPLITS = ((0, 64), (64, 61))
PB = 64


def _sc_body(g_hbm, d_hbm, xall_hbm, out_hbm,
             gbuf, dbuf, rows0, rows1, zbuf, agg, sem0, sem1):
    cid = lax.axis_index("c")
    sid = lax.axis_index("s")
    wid = cid * NS + sid

    def _zr(i, carry):
        for j in range(D // LANES):
            zbuf[i, pl.ds(j * LANES, LANES)] = jnp.zeros((LANES,), jnp.float32)
        return carry

    lax.fori_loop(0, DC, _zr, 0)
    for k in range(SR // DC):
        pltpu.sync_copy(zbuf, agg.at[pl.ds(sid * SR + k * DC, DC)])
    plsc.subcore_barrier()

    rows = (rows0, rows1)
    sems = (sem0, sem1)

    def _gather(c, p):
        pltpu.make_async_copy(xall_hbm.at[gbuf.at[c]], rows[p], sems[p]).start()

    def _drain(c, p):
        pltpu.make_async_copy(xall_hbm.at[gbuf.at[c]], rows[p], sems[p]).wait()

    def _scatter(c, p):
        pltpu.sync_copy(rows[p], agg.at[dbuf.at[c]], add=True)

    for base, n in PASS_SPLITS:
        pltpu.sync_copy(g_hbm.at[wid, pl.ds(base, n)], gbuf.at[pl.ds(0, n)])
        pltpu.sync_copy(d_hbm.at[1, wid, pl.ds(base, n)], dbuf.at[pl.ds(0, n)])
        _gather(0, 0)
        for c in range(1, n):
            _gather(c, c % 2)
            _drain(c - 1, (c - 1) % 2)
            _scatter(c - 1, (c - 1) % 2)
        _drain(n - 1, (n - 1) % 2)
        _scatter(n - 1, (n - 1) % 2)

    plsc.subcore_barrier()

    r0 = sid * SR
    pltpu.sync_copy(agg.at[pl.ds(r0, SR)], out_hbm.at[pl.ds(cid * NP + r0, SR)])


def _sc_agg(g3, d3, xall_flat):
    mesh = plsc.VectorSubcoreMesh(core_axis_name="c", subcore_axis_name="s")
    kern = pl.kernel(
        _sc_body,
        mesh=mesh,
        out_type=jax.ShapeDtypeStruct((NC * NP, D), jnp.float32),
        scratch_types=[
            pltpu.VMEM((PB, CH), jnp.int32),
            pltpu.VMEM((PB, CH), jnp.int32),
            pltpu.VMEM((CH, D), jnp.float32),
            pltpu.VMEM((CH, D), jnp.float32),
            pltpu.VMEM((DC, D), jnp.float32),
            pltpu.VMEM_SHARED((NP, D), jnp.float32),
            pltpu.SemaphoreType.DMA,
            pltpu.SemaphoreType.DMA,
        ],
    )
    return kern(g3, d3, xall_flat)


BO = 80


def _out2_body(a0_ref, a1_ref, x_ref, root_ref, bias_ref, ir_ref, wr_ref,
               o_ref, rel_ref):
    rel_ref[...] = jnp.dot(ir_ref[...], wr_ref[...],
                           preferred_element_type=jnp.float32)
    o_ref[...] = (a0_ref[0:N] + a1_ref[0:N] + bias_ref[...]
                  + jnp.dot(x_ref[...], root_ref[...],
                            preferred_element_type=jnp.float32))


def _out2(agg2, x, root, bias, init_rel, w_rel):
    return pl.pallas_call(
        _out2_body,
        grid=(1,),
        in_specs=[
            pl.BlockSpec((NP, D), lambda i: (0, 0)),
            pl.BlockSpec((NP, D), lambda i: (1, 0)),
            pl.BlockSpec((N, D), lambda i: (0, 0)),
            pl.BlockSpec((D, D), lambda i: (0, 0)),
            pl.BlockSpec((1, D), lambda i: (0, 0)),
            pl.BlockSpec((2 * R, D), lambda i: (0, 0)),
            pl.BlockSpec((D, D), lambda i: (0, 0)),
        ],
        out_specs=[
            pl.BlockSpec((N, D), lambda i: (0, 0)),
            pl.BlockSpec((2 * R, D), lambda i: (0, 0)),
        ],
        out_shape=[
            jax.ShapeDtypeStruct((N, D), jnp.float32),
            jax.ShapeDtypeStruct((2 * R, D), jnp.float32),
        ],
    )(agg2, agg2, x, root, bias.reshape(1, D), init_rel, w_rel)


def kernel(edge_index, edge_type, init_embed, init_rel, w_rel,
           basis1, comp1, root1, bias1, basis2, comp2, root2, bias2):
    t2 = edge_type.reshape(NW * NCH, CH)
    ei2 = edge_index.reshape(2, NW * NCH, CH)
    d4 = edge_index.reshape(2, NW, NCH, CH)

    xall1, g2 = _xall1(init_embed, basis1, comp1, t2, ei2)
    g3 = g2.reshape(NW, NCH, CH)
    agg1 = _sc_agg(g3, d4, xall1.reshape(R * N, D))
    xall2, x1 = _xall2(agg1, init_embed, root1, bias1, basis2, comp2)
    agg2 = _sc_agg(g3, d4, xall2.reshape(R * N, D))
    x2, rel = _out2(agg2, x1, root2, bias2, init_rel, w_rel)
    return (x2, rel)
